# baseline (speedup 1.0000x reference)
import functools

import jax
import jax.numpy as jnp
from jax import lax
from jax.experimental import pallas as pl
from jax.experimental.pallas import tpu as pltpu
import jax.experimental.pallas.tpu_sc as plsc

NUM_CH = 2048
ROWS = 4 * 4096
LANES = 16
NC, NS = 2, 16
NW = NC * NS
ROWS_PER_W = ROWS // NW
G = 8
NG = ROWS_PER_W // G
GW = G * NUM_CH


def _sc_rev_body(x_hbm, out_hbm, inbuf, outbuf, si0, si1, so0, so1):
    wid = lax.axis_index("s") * NC + lax.axis_index("c")
    base = wid * ROWS_PER_W * NUM_CH
    in_sems = (si0, si1)
    out_sems = (so0, so1)

    def in_copy(g, slot):
        return pltpu.make_async_copy(
            x_hbm.at[pl.ds(base + g * GW, GW)], inbuf.at[slot], in_sems[slot])

    def out_copy(g, slot):
        return pltpu.make_async_copy(
            outbuf.at[slot], out_hbm.at[pl.ds(base + g * GW, GW)],
            out_sems[slot])

    in_copy(0, 0).start()
    in_copy(1, 1).start()

    def step(i, carry):
        for slot in (0, 1):
            g = 2 * i + slot
            in_copy(g, slot).wait()

            @pl.when(i > 0)
            def _():
                out_copy(g - 2, slot).wait()

            @functools.partial(plsc.parallel_loop, 0, G, unroll=2)
            def _(r):
                rb = r * NUM_CH
                for c in range(NUM_CH // LANES):
                    src = inbuf[slot,
                                pl.ds(rb + NUM_CH - LANES * (c + 1), LANES)]
                    outbuf[slot, pl.ds(rb + LANES * c, LANES)] = (
                        lax.rev(src, (0,)))

            out_copy(g, slot).start()

            @pl.when(g + 2 < NG)
            def _():
                in_copy(g + 2, slot).start()
        return carry

    lax.fori_loop(0, NG // 2, step, 0)
    out_copy(NG - 2, 0).wait()
    out_copy(NG - 1, 1).wait()


@functools.lru_cache(maxsize=1)
def _sc_rev_call():
    return pl.kernel(
        _sc_rev_body,
        out_type=jax.ShapeDtypeStruct((ROWS * NUM_CH,), jnp.float32),
        mesh=plsc.VectorSubcoreMesh(
            core_axis_name="c", subcore_axis_name="s",
            num_cores=NC, num_subcores=NS),
        scratch_types=[
            pltpu.VMEM((2, GW), jnp.float32),
            pltpu.VMEM((2, GW), jnp.float32),
            pltpu.SemaphoreType.DMA,
            pltpu.SemaphoreType.DMA,
            pltpu.SemaphoreType.DMA,
            pltpu.SemaphoreType.DMA,
        ],
    )


def kernel(input):
    x = input.reshape(ROWS * NUM_CH)
    out = _sc_rev_call()(x)
    return out.reshape(input.shape)


# speedup vs baseline: 2.4980x; 2.4980x over previous
import functools

import jax
import jax.numpy as jnp
from jax import lax
from jax.experimental import pallas as pl
from jax.experimental.pallas import tpu as pltpu
import jax.experimental.pallas.tpu_sc as plsc

NUM_CH = 2048
ROWS = 4 * 4096
LANES = 16
NC, NS = 2, 16
NW = NC * NS
ROWS_PER_W = ROWS // NW
G = 4
NG = ROWS_PER_W // G


def _sc_rev_body(x_hbm, out_hbm, inbuf, outbuf, si0, si1, so0, so1):
    wid = lax.axis_index("s") * NC + lax.axis_index("c")
    base = wid * ROWS_PER_W
    in_sems = (si0, si1)
    out_sems = (so0, so1)

    def in_copy(g, slot):
        return pltpu.make_async_copy(
            x_hbm.at[pl.ds(base + g * G, G)], inbuf.at[slot], in_sems[slot])

    def out_copy(g, slot):
        return pltpu.make_async_copy(
            outbuf.at[slot], out_hbm.at[pl.ds(base + g * G, G)], out_sems[slot])

    in_copy(0, 0).start()
    in_copy(1, 1).start()

    def step(i, carry):
        for slot in (0, 1):
            g = 2 * i + slot
            in_copy(g, slot).wait()

            @pl.when(i > 0)
            def _():
                out_copy(g - 2, slot).wait()

            for r in range(G):
                for c in range(NUM_CH // LANES):
                    src = inbuf[slot, r,
                                pl.ds(NUM_CH - LANES * (c + 1), LANES)]
                    outbuf[slot, r, pl.ds(LANES * c, LANES)] = (
                        lax.rev(src, (0,)))
                pltpu.make_async_copy(
                    outbuf.at[slot, pl.ds(r, 1)],
                    out_hbm.at[pl.ds(base + g * G + r, 1)],
                    out_sems[slot]).start()

            @pl.when(g + 2 < NG)
            def _():
                in_copy(g + 2, slot).start()
        return carry

    lax.fori_loop(0, NG // 2, step, 0)
    out_copy(NG - 2, 0).wait()
    out_copy(NG - 1, 1).wait()


@functools.lru_cache(maxsize=1)
def _sc_rev_call():
    return pl.kernel(
        _sc_rev_body,
        out_type=jax.ShapeDtypeStruct((ROWS, NUM_CH), jnp.float32),
        mesh=plsc.VectorSubcoreMesh(
            core_axis_name="c", subcore_axis_name="s",
            num_cores=NC, num_subcores=NS),
        scratch_types=[
            pltpu.VMEM((2, G, NUM_CH), jnp.float32),
            pltpu.VMEM((2, G, NUM_CH), jnp.float32),
            pltpu.SemaphoreType.DMA,
            pltpu.SemaphoreType.DMA,
            pltpu.SemaphoreType.DMA,
            pltpu.SemaphoreType.DMA,
        ],
    )


def kernel(input):
    x = input.reshape(ROWS, NUM_CH)
    out = _sc_rev_call()(x)
    return out.reshape(input.shape)


# speedup vs baseline: 2.5992x; 1.0405x over previous
import functools

import jax
import jax.numpy as jnp
from jax import lax
from jax.experimental import pallas as pl
from jax.experimental.pallas import tpu as pltpu
import jax.experimental.pallas.tpu_sc as plsc

NUM_CH = 2048
ROWS = 4 * 4096
LANES = 16
NC, NS = 2, 16
NW = NC * NS
ROWS_PER_W = ROWS // NW
G = 4
NG = ROWS_PER_W // G


def _sc_rev_body(x_hbm, out_hbm, inbuf, outbuf, si0, si1, so0, so1):
    wid = lax.axis_index("s") * NC + lax.axis_index("c")
    base = wid * ROWS_PER_W
    in_sems = (si0, si1)
    out_sems = (so0, so1)

    def in_copy(g, slot):
        return pltpu.make_async_copy(
            x_hbm.at[pl.ds(base + g * G, G)], inbuf.at[slot], in_sems[slot])

    def out_copy(g, slot):
        return pltpu.make_async_copy(
            outbuf.at[slot], out_hbm.at[pl.ds(base + g * G, G)], out_sems[slot])

    in_copy(0, 0).start()
    in_copy(1, 1).start()

    def step(i, carry):
        for slot in (0, 1):
            g = 2 * i + slot
            in_copy(g, slot).wait()

            @pl.when(i > 0)
            def _():
                out_copy(g - 2, slot).wait()

            for r in range(G):
                for c in range(NUM_CH // LANES):
                    src = inbuf[slot, r,
                                pl.ds(NUM_CH - LANES * (c + 1), LANES)]
                    outbuf[slot, r, pl.ds(LANES * c, LANES)] = (
                        lax.rev(src, (0,)))

            out_copy(g, slot).start()

            @pl.when(g + 2 < NG)
            def _():
                in_copy(g + 2, slot).start()
        return carry

    lax.fori_loop(0, NG // 2, step, 0)
    out_copy(NG - 2, 0).wait()
    out_copy(NG - 1, 1).wait()


@functools.lru_cache(maxsize=1)
def _sc_rev_call():
    return pl.kernel(
        _sc_rev_body,
        out_type=jax.ShapeDtypeStruct((ROWS, NUM_CH), jnp.float32),
        mesh=plsc.VectorSubcoreMesh(
            core_axis_name="c", subcore_axis_name="s",
            num_cores=NC, num_subcores=NS),
        scratch_types=[
            pltpu.VMEM((2, G, NUM_CH), jnp.float32),
            pltpu.VMEM((2, G, NUM_CH), jnp.float32),
            pltpu.SemaphoreType.DMA,
            pltpu.SemaphoreType.DMA,
            pltpu.SemaphoreType.DMA,
            pltpu.SemaphoreType.DMA,
        ],
    )


def kernel(input):
    x = input.reshape(ROWS, NUM_CH)
    out = _sc_rev_call()(x)
    return out.reshape(input.shape)
